# baseline (speedup 1.0000x reference)
import functools

import jax
import jax.numpy as jnp
from jax import lax
from jax.experimental import pallas as pl
from jax.experimental.pallas import tpu as pltpu
from jax.experimental.pallas import tpu_sc as plsc

_N = 2048
_K = 150
_NCOMP = 784
_BLK = 256
_L = 16


def _knn_select_body(d2_ref, a_ref):
    nrow = d2_ref.shape[0]
    d2 = d2_ref[...]
    bits = jax.lax.bitcast_convert_type(d2, jnp.int32)
    skey = jnp.where(bits < 0, bits ^ jnp.int32(0x7FFFFFFF), bits)

    k0 = jnp.full((nrow, 1), _K, jnp.int32)
    cnt_neg = jnp.sum((skey < 0).astype(jnp.int32), axis=1, keepdims=True)
    use_neg = k0 <= cnt_neg
    prefix = jnp.where(use_neg, jnp.int32(-(2**31)), jnp.int32(0))
    k_rem = jnp.where(use_neg, k0, k0 - cnt_neg)

    for b in range(30, -1, -1):
        match0 = (skey >> b) == (prefix >> b)
        cnt0 = jnp.sum(match0.astype(jnp.int32), axis=1, keepdims=True)
        take1 = k_rem > cnt0
        prefix = jnp.where(take1, prefix | jnp.int32(1 << b), prefix)
        k_rem = jnp.where(take1, k_rem - cnt0, k_rem)

    t = prefix
    lt = skey < t
    tie = skey == t
    need = k0 - jnp.sum(lt.astype(jnp.int32), axis=1, keepdims=True)
    col = jax.lax.broadcasted_iota(jnp.int32, (nrow, _N), 1)
    ipref = jnp.zeros((nrow, 1), jnp.int32)
    for b in range(10, -1, -1):
        m0 = tie & ((col >> b) == (ipref >> b))
        cnt0 = jnp.sum(m0.astype(jnp.int32), axis=1, keepdims=True)
        take1 = need > cnt0
        ipref = jnp.where(take1, ipref | jnp.int32(1 << b), ipref)
        need = jnp.where(take1, need - cnt0, need)

    sel = lt | (tie & (col <= ipref))
    a_ref[...] = sel.astype(jnp.float32)


def _knn_adjacency_tc(d2_rows, interpret=False):
    nrows = d2_rows.shape[0]
    blk = min(_BLK, nrows)
    return pl.pallas_call(
        _knn_select_body,
        grid=(nrows // blk,),
        in_specs=[pl.BlockSpec((blk, _N), lambda i: (i, 0))],
        out_specs=pl.BlockSpec((blk, _N), lambda i: (i, 0)),
        out_shape=jax.ShapeDtypeStruct((nrows, _N), jnp.float32),
        interpret=interpret,
    )(d2_rows)


def _sc_knn_adjacency(d2_flat, s_rows, row_base):
    info = plsc.get_sparse_core_info()
    nw = info.num_cores * info.num_subcores
    rp = s_rows // nw
    batch = min(16, rp)
    nb = rp // batch
    nv = _N // _L

    mesh = plsc.VectorSubcoreMesh(core_axis_name="c", subcore_axis_name="s")

    @functools.partial(
        pl.kernel, mesh=mesh,
        out_type=jax.ShapeDtypeStruct((s_rows * _N,), jnp.float32),
        scratch_types=[
            pltpu.VMEM((batch * _N,), jnp.float32),
            pltpu.VMEM((batch * _N,), jnp.float32),
            pltpu.VMEM((_N,), jnp.int32),
        ],
    )
    def sc_kernel(d2_hbm, a_hbm, in_v, out_v, skey_v):
        wid = lax.axis_index("s") * info.num_cores + lax.axis_index("c")
        wbase = wid * rp
        iota = lax.broadcasted_iota(jnp.int32, (_L,), 0)

        def hsum(v):
            parts = [v[i] for i in range(_L)]
            while len(parts) > 1:
                parts = [parts[i] + parts[i + 1]
                         for i in range(0, len(parts), 2)]
            return parts[0]

        def count_pass(pred):
            def cb(j, acc):
                v = skey_v[pl.ds(j * _L, _L)]
                return acc + jnp.where(pred(v, j), 1, 0).astype(jnp.int32)
            acc = lax.fori_loop(0, nv, cb, jnp.zeros((_L,), jnp.int32))
            return hsum(acc)

        def row_body(ri, _):
            def p0(j, carry):
                vmn, vmx, vneg = carry
                v = in_v[pl.ds(ri * _N + j * _L, _L)]
                b = lax.bitcast_convert_type(v, jnp.int32)
                sk = jnp.where(b < 0, b ^ jnp.int32(0x7FFFFFFF), b)
                skey_v[pl.ds(j * _L, _L)] = sk
                return (jnp.minimum(vmn, sk), jnp.maximum(vmx, sk),
                        vneg + jnp.where(sk < 0, 1, 0).astype(jnp.int32))
            vmn, vmx, vneg = lax.fori_loop(
                0, nv, p0,
                (jnp.full((_L,), 2**31 - 1, jnp.int32),
                 jnp.full((_L,), -(2**31), jnp.int32),
                 jnp.zeros((_L,), jnp.int32)))
            mn, mx = vmn[0], vmx[0]
            for i in range(1, _L):
                mn = jnp.minimum(mn, vmn[i])
                mx = jnp.maximum(mx, vmx[i])
            cneg = hsum(vneg)

            x = mn ^ mx
            fx = x.astype(jnp.float32)
            e = (lax.bitcast_convert_type(fx, jnp.int32) >> 23) - 127
            mixed = x < 0
            h = jnp.where(mixed, jnp.int32(31), e)

            k0 = jnp.int32(_K)
            use_neg = mixed & (k0 <= cneg)
            s = jnp.clip(h + 1, 0, 31)
            mask_hi = jnp.where(h >= 31, jnp.int32(0), jnp.int32(-1) << s)
            prefix0 = jnp.where(
                mixed,
                jnp.where(use_neg, jnp.int32(-(2**31)), jnp.int32(0)),
                mn & mask_hi)
            krem0 = jnp.where(mixed & jnp.logical_not(use_neg),
                              k0 - cneg, k0)
            nbits = jnp.where(mixed, jnp.int32(31),
                              jnp.maximum(h + 1, 0))

            def vbody(i, carry):
                prefix, krem = carry
                b = nbits - 1 - i
                ps = prefix >> b
                cnt0 = count_pass(lambda v, j: (v >> b) == ps)
                take1 = krem > cnt0
                prefix = jnp.where(take1, prefix | (jnp.int32(1) << b),
                                   prefix)
                krem = jnp.where(take1, krem - cnt0, krem)
                return prefix, krem
            t, _unused = lax.fori_loop(0, nbits, vbody, (prefix0, krem0))

            def c2(j, carry):
                a_lt, a_eq = carry
                v = skey_v[pl.ds(j * _L, _L)]
                a_lt = a_lt + jnp.where(v < t, 1, 0).astype(jnp.int32)
                a_eq = a_eq + jnp.where(v == t, 1, 0).astype(jnp.int32)
                return a_lt, a_eq
            a_lt, a_eq = lax.fori_loop(
                0, nv, c2,
                (jnp.zeros((_L,), jnp.int32), jnp.zeros((_L,), jnp.int32)))
            cnt_lt = hsum(a_lt)
            cnt_eq = hsum(a_eq)
            need = k0 - cnt_lt

            def sel_idx(_op):
                def ib(i, carry):
                    ipref, nd = carry
                    b = jnp.int32(10) - i
                    ps = ipref >> b
                    cnt0 = count_pass(
                        lambda v, j: (v == t)
                        & (((iota + j * _L) >> b) == ps))
                    take1 = nd > cnt0
                    ipref = jnp.where(take1, ipref | (jnp.int32(1) << b),
                                      ipref)
                    nd = jnp.where(take1, nd - cnt0, nd)
                    return ipref, nd
                ipref, _nd = lax.fori_loop(0, 11, ib, (jnp.int32(0), need))
                return ipref
            ipref = lax.cond(cnt_eq == need,
                             lambda _op: jnp.int32(_N - 1), sel_idx, 0)

            def eb(j, _c):
                v = skey_v[pl.ds(j * _L, _L)]
                col = iota + j * _L
                sel = (v < t) | ((v == t) & (col <= ipref))
                out_v[pl.ds(ri * _N + j * _L, _L)] = jnp.where(
                    sel, jnp.float32(1.0), jnp.float32(0.0))
                return 0
            lax.fori_loop(0, nv, eb, 0)
            return 0

        def batch_body(bi, _):
            gbase = wbase + bi * batch
            pltpu.sync_copy(
                d2_hbm.at[pl.ds((row_base + gbase) * _N, batch * _N)], in_v)
            lax.fori_loop(0, batch, row_body, 0)
            pltpu.sync_copy(out_v, a_hbm.at[pl.ds(gbase * _N, batch * _N)])
            return 0
        lax.fori_loop(0, nb, batch_body, 0)

    return sc_kernel(d2_flat).reshape(s_rows, _N)


_S_SC = 2048


def kernel(toLearn):
    flat = toLearn.reshape(toLearn.shape[0], -1)
    n = flat.shape[0]
    sq = jnp.sum(flat * flat, axis=1)
    d2 = sq[:, None] + sq[None, :] - 2.0 * (flat @ flat.T)
    d2 = d2 + jnp.eye(n, dtype=flat.dtype) * 1e12

    parts = []
    if _S_SC > 0:
        parts.append(_sc_knn_adjacency(d2.reshape(-1), _S_SC, 0))
    if _S_SC < n:
        parts.append(_knn_adjacency_tc(d2[_S_SC:]))
    A = parts[0] if len(parts) == 1 else jnp.concatenate(parts, axis=0)

    W = 0.5 * (A + A.T)
    deg = jnp.sum(W, axis=1)
    dd = jnp.sqrt(deg)
    L = jnp.eye(n, dtype=jnp.float32) - (W / dd[:, None]) / dd[None, :]
    evals, evecs = jnp.linalg.eigh(L)
    emb = evecs[:, 1:_NCOMP + 1] / dd[:, None]
    max_abs_row = jnp.argmax(jnp.abs(emb), axis=0)
    signs = jnp.sign(emb[max_abs_row, jnp.arange(emb.shape[1])])
    signs = jnp.where(signs == 0, 1.0, signs)
    emb = jax.lax.stop_gradient(emb * signs[None, :])
    return emb.reshape(n, 1, 28, 28).astype(jnp.float32)


# speedup vs baseline: 1.0137x; 1.0137x over previous
import functools

import jax
import jax.numpy as jnp
from jax import lax
from jax.experimental import pallas as pl
from jax.experimental.pallas import tpu as pltpu
from jax.experimental.pallas import tpu_sc as plsc

_N = 2048
_K = 150
_NCOMP = 784
_BLK = 256
_L = 16


def _knn_select_body(d2_ref, a_ref):
    nrow = d2_ref.shape[0]
    d2 = d2_ref[...]
    bits = jax.lax.bitcast_convert_type(d2, jnp.int32)
    skey = jnp.where(bits < 0, bits ^ jnp.int32(0x7FFFFFFF), bits)

    k0 = jnp.full((nrow, 1), _K, jnp.int32)
    cnt_neg = jnp.sum((skey < 0).astype(jnp.int32), axis=1, keepdims=True)
    use_neg = k0 <= cnt_neg
    prefix = jnp.where(use_neg, jnp.int32(-(2**31)), jnp.int32(0))
    k_rem = jnp.where(use_neg, k0, k0 - cnt_neg)

    for b in range(30, -1, -1):
        match0 = (skey >> b) == (prefix >> b)
        cnt0 = jnp.sum(match0.astype(jnp.int32), axis=1, keepdims=True)
        take1 = k_rem > cnt0
        prefix = jnp.where(take1, prefix | jnp.int32(1 << b), prefix)
        k_rem = jnp.where(take1, k_rem - cnt0, k_rem)

    t = prefix
    lt = skey < t
    tie = skey == t
    need = k0 - jnp.sum(lt.astype(jnp.int32), axis=1, keepdims=True)
    col = jax.lax.broadcasted_iota(jnp.int32, (nrow, _N), 1)
    ipref = jnp.zeros((nrow, 1), jnp.int32)
    for b in range(10, -1, -1):
        m0 = tie & ((col >> b) == (ipref >> b))
        cnt0 = jnp.sum(m0.astype(jnp.int32), axis=1, keepdims=True)
        take1 = need > cnt0
        ipref = jnp.where(take1, ipref | jnp.int32(1 << b), ipref)
        need = jnp.where(take1, need - cnt0, need)

    sel = lt | (tie & (col <= ipref))
    a_ref[...] = sel.astype(jnp.float32)


def _knn_adjacency_tc(d2_rows, interpret=False):
    nrows = d2_rows.shape[0]
    blk = min(_BLK, nrows)
    return pl.pallas_call(
        _knn_select_body,
        grid=(nrows // blk,),
        in_specs=[pl.BlockSpec((blk, _N), lambda i: (i, 0))],
        out_specs=pl.BlockSpec((blk, _N), lambda i: (i, 0)),
        out_shape=jax.ShapeDtypeStruct((nrows, _N), jnp.float32),
        interpret=interpret,
    )(d2_rows)


def _sc_knn_adjacency(d2_flat, s_rows, row_base):
    info = plsc.get_sparse_core_info()
    nw = info.num_cores * info.num_subcores
    rp = s_rows // nw
    batch = min(16, rp)
    nb = rp // batch
    nv = _N // _L

    mesh = plsc.VectorSubcoreMesh(core_axis_name="c", subcore_axis_name="s")

    @functools.partial(
        pl.kernel, mesh=mesh,
        out_type=jax.ShapeDtypeStruct((s_rows * _N,), jnp.float32),
        scratch_types=[
            pltpu.VMEM((batch * _N,), jnp.float32),
            pltpu.VMEM((batch * _N,), jnp.float32),
            pltpu.VMEM((_N,), jnp.int32),
        ],
    )
    def sc_kernel(d2_hbm, a_hbm, in_v, out_v, skey_v):
        wid = lax.axis_index("s") * info.num_cores + lax.axis_index("c")
        wbase = wid * rp
        iota = lax.broadcasted_iota(jnp.int32, (_L,), 0)

        def hsum(v):
            parts = [v[i] for i in range(_L)]
            while len(parts) > 1:
                parts = [parts[i] + parts[i + 1]
                         for i in range(0, len(parts), 2)]
            return parts[0]

        def count_pass(pred):
            def cb(j, acc):
                v = skey_v[pl.ds(j * _L, _L)]
                return acc + jnp.where(pred(v, j), 1, 0).astype(jnp.int32)
            acc = lax.fori_loop(0, nv, cb, jnp.zeros((_L,), jnp.int32))
            return hsum(acc)

        def row_body(ri, _):
            def p0(j, carry):
                vmn, vmx, vneg = carry
                v = in_v[pl.ds(ri * _N + j * _L, _L)]
                b = lax.bitcast_convert_type(v, jnp.int32)
                sk = jnp.where(b < 0, b ^ jnp.int32(0x7FFFFFFF), b)
                skey_v[pl.ds(j * _L, _L)] = sk
                return (jnp.minimum(vmn, sk), jnp.maximum(vmx, sk),
                        vneg + jnp.where(sk < 0, 1, 0).astype(jnp.int32))
            vmn, vmx, vneg = lax.fori_loop(
                0, nv, p0,
                (jnp.full((_L,), 2**31 - 1, jnp.int32),
                 jnp.full((_L,), -(2**31), jnp.int32),
                 jnp.zeros((_L,), jnp.int32)))
            mn, mx = vmn[0], vmx[0]
            for i in range(1, _L):
                mn = jnp.minimum(mn, vmn[i])
                mx = jnp.maximum(mx, vmx[i])
            cneg = hsum(vneg)

            x = mn ^ mx
            fx = x.astype(jnp.float32)
            e = (lax.bitcast_convert_type(fx, jnp.int32) >> 23) - 127
            mixed = x < 0
            h = jnp.where(mixed, jnp.int32(31), e)

            k0 = jnp.int32(_K)
            use_neg = mixed & (k0 <= cneg)
            s = jnp.clip(h + 1, 0, 31)
            mask_hi = jnp.where(h >= 31, jnp.int32(0), jnp.int32(-1) << s)
            prefix0 = jnp.where(
                mixed,
                jnp.where(use_neg, jnp.int32(-(2**31)), jnp.int32(0)),
                mn & mask_hi)
            krem0 = jnp.where(mixed & jnp.logical_not(use_neg),
                              k0 - cneg, k0)
            nbits = jnp.where(mixed, jnp.int32(31),
                              jnp.maximum(h + 1, 0))

            def vbody(i, carry):
                prefix, krem = carry
                b = nbits - 1 - i
                ps = prefix >> b
                cnt0 = count_pass(lambda v, j: (v >> b) == ps)
                take1 = krem > cnt0
                prefix = jnp.where(take1, prefix | (jnp.int32(1) << b),
                                   prefix)
                krem = jnp.where(take1, krem - cnt0, krem)
                return prefix, krem
            t, _unused = lax.fori_loop(0, nbits, vbody, (prefix0, krem0))

            def c2(j, carry):
                a_lt, a_eq = carry
                v = skey_v[pl.ds(j * _L, _L)]
                a_lt = a_lt + jnp.where(v < t, 1, 0).astype(jnp.int32)
                a_eq = a_eq + jnp.where(v == t, 1, 0).astype(jnp.int32)
                return a_lt, a_eq
            a_lt, a_eq = lax.fori_loop(
                0, nv, c2,
                (jnp.zeros((_L,), jnp.int32), jnp.zeros((_L,), jnp.int32)))
            cnt_lt = hsum(a_lt)
            cnt_eq = hsum(a_eq)
            need = k0 - cnt_lt

            def sel_idx(_op):
                def ib(i, carry):
                    ipref, nd = carry
                    b = jnp.int32(10) - i
                    ps = ipref >> b
                    cnt0 = count_pass(
                        lambda v, j: (v == t)
                        & (((iota + j * _L) >> b) == ps))
                    take1 = nd > cnt0
                    ipref = jnp.where(take1, ipref | (jnp.int32(1) << b),
                                      ipref)
                    nd = jnp.where(take1, nd - cnt0, nd)
                    return ipref, nd
                ipref, _nd = lax.fori_loop(0, 11, ib, (jnp.int32(0), need))
                return ipref
            ipref = lax.cond(cnt_eq == need,
                             lambda _op: jnp.int32(_N - 1), sel_idx, 0)

            def eb(j, _c):
                v = skey_v[pl.ds(j * _L, _L)]
                col = iota + j * _L
                sel = (v < t) | ((v == t) & (col <= ipref))
                out_v[pl.ds(ri * _N + j * _L, _L)] = jnp.where(
                    sel, jnp.float32(1.0), jnp.float32(0.0))
                return 0
            lax.fori_loop(0, nv, eb, 0)
            return 0

        def batch_body(bi, _):
            gbase = wbase + bi * batch
            pltpu.sync_copy(
                d2_hbm.at[pl.ds((row_base + gbase) * _N, batch * _N)], in_v)
            lax.fori_loop(0, batch, row_body, 0)
            pltpu.sync_copy(out_v, a_hbm.at[pl.ds(gbase * _N, batch * _N)])
            return 0
        lax.fori_loop(0, nb, batch_body, 0)

    return sc_kernel(d2_flat).reshape(s_rows, _N)


_S_SC = 512


def kernel(toLearn):
    flat = toLearn.reshape(toLearn.shape[0], -1)
    n = flat.shape[0]
    sq = jnp.sum(flat * flat, axis=1)
    d2 = sq[:, None] + sq[None, :] - 2.0 * (flat @ flat.T)
    d2 = d2 + jnp.eye(n, dtype=flat.dtype) * 1e12

    parts = []
    if _S_SC > 0:
        parts.append(_sc_knn_adjacency(d2.reshape(-1), _S_SC, 0))
    if _S_SC < n:
        parts.append(_knn_adjacency_tc(d2[_S_SC:]))
    A = parts[0] if len(parts) == 1 else jnp.concatenate(parts, axis=0)

    W = 0.5 * (A + A.T)
    deg = jnp.sum(W, axis=1)
    dd = jnp.sqrt(deg)
    L = jnp.eye(n, dtype=jnp.float32) - (W / dd[:, None]) / dd[None, :]
    evals, evecs = jnp.linalg.eigh(L)
    emb = evecs[:, 1:_NCOMP + 1] / dd[:, None]
    max_abs_row = jnp.argmax(jnp.abs(emb), axis=0)
    signs = jnp.sign(emb[max_abs_row, jnp.arange(emb.shape[1])])
    signs = jnp.where(signs == 0, 1.0, signs)
    emb = jax.lax.stop_gradient(emb * signs[None, :])
    return emb.reshape(n, 1, 28, 28).astype(jnp.float32)


# speedup vs baseline: 1.0169x; 1.0032x over previous
import functools

import jax
import jax.numpy as jnp
from jax import lax
from jax.experimental import pallas as pl
from jax.experimental.pallas import tpu as pltpu
from jax.experimental.pallas import tpu_sc as plsc

_N = 2048
_K = 150
_NCOMP = 784
_BLK = 256
_L = 16


def _knn_select_body(d2_ref, a_ref):
    nrow = d2_ref.shape[0]
    d2 = d2_ref[...]
    bits = jax.lax.bitcast_convert_type(d2, jnp.int32)
    skey = jnp.where(bits < 0, bits ^ jnp.int32(0x7FFFFFFF), bits)

    k0 = jnp.full((nrow, 1), _K, jnp.int32)
    cnt_neg = jnp.sum((skey < 0).astype(jnp.int32), axis=1, keepdims=True)
    use_neg = k0 <= cnt_neg
    prefix = jnp.where(use_neg, jnp.int32(-(2**31)), jnp.int32(0))
    k_rem = jnp.where(use_neg, k0, k0 - cnt_neg)

    for b in range(30, -1, -1):
        match0 = (skey >> b) == (prefix >> b)
        cnt0 = jnp.sum(match0.astype(jnp.int32), axis=1, keepdims=True)
        take1 = k_rem > cnt0
        prefix = jnp.where(take1, prefix | jnp.int32(1 << b), prefix)
        k_rem = jnp.where(take1, k_rem - cnt0, k_rem)

    t = prefix
    lt = skey < t
    tie = skey == t
    need = k0 - jnp.sum(lt.astype(jnp.int32), axis=1, keepdims=True)
    col = jax.lax.broadcasted_iota(jnp.int32, (nrow, _N), 1)
    ipref = jnp.zeros((nrow, 1), jnp.int32)
    for b in range(10, -1, -1):
        m0 = tie & ((col >> b) == (ipref >> b))
        cnt0 = jnp.sum(m0.astype(jnp.int32), axis=1, keepdims=True)
        take1 = need > cnt0
        ipref = jnp.where(take1, ipref | jnp.int32(1 << b), ipref)
        need = jnp.where(take1, need - cnt0, need)

    sel = lt | (tie & (col <= ipref))
    a_ref[...] = sel.astype(jnp.float32)


def _knn_adjacency_tc(d2_rows, interpret=False):
    nrows = d2_rows.shape[0]
    blk = min(_BLK, nrows)
    return pl.pallas_call(
        _knn_select_body,
        grid=(nrows // blk,),
        in_specs=[pl.BlockSpec((blk, _N), lambda i: (i, 0))],
        out_specs=pl.BlockSpec((blk, _N), lambda i: (i, 0)),
        out_shape=jax.ShapeDtypeStruct((nrows, _N), jnp.float32),
        interpret=interpret,
    )(d2_rows)


def _sc_knn_adjacency(d2_flat, s_rows, row_base):
    info = plsc.get_sparse_core_info()
    nw = info.num_cores * info.num_subcores
    rp = s_rows // nw
    batch = min(16, rp)
    nb = rp // batch
    nv = _N // _L

    mesh = plsc.VectorSubcoreMesh(core_axis_name="c", subcore_axis_name="s")

    @functools.partial(
        pl.kernel, mesh=mesh,
        out_type=jax.ShapeDtypeStruct((s_rows * _N,), jnp.float32),
        scratch_types=[
            pltpu.VMEM((batch * _N,), jnp.float32),
            pltpu.VMEM((batch * _N,), jnp.float32),
            pltpu.VMEM((_N,), jnp.int32),
        ],
    )
    def sc_kernel(d2_hbm, a_hbm, in_v, out_v, skey_v):
        wid = lax.axis_index("s") * info.num_cores + lax.axis_index("c")
        wbase = wid * rp
        iota = lax.broadcasted_iota(jnp.int32, (_L,), 0)

        def hsum(v):
            parts = [v[i] for i in range(_L)]
            while len(parts) > 1:
                parts = [parts[i] + parts[i + 1]
                         for i in range(0, len(parts), 2)]
            return parts[0]

        def count_pass(pred):
            def cb(j, acc):
                v = skey_v[pl.ds(j * _L, _L)]
                return acc + jnp.where(pred(v, j), 1, 0).astype(jnp.int32)
            acc = lax.fori_loop(0, nv, cb, jnp.zeros((_L,), jnp.int32),
                                unroll=4)
            return hsum(acc)

        def row_body(ri, _):
            def p0(j, carry):
                vmn, vmx, vneg = carry
                v = in_v[pl.ds(ri * _N + j * _L, _L)]
                b = lax.bitcast_convert_type(v, jnp.int32)
                sk = jnp.where(b < 0, b ^ jnp.int32(0x7FFFFFFF), b)
                skey_v[pl.ds(j * _L, _L)] = sk
                return (jnp.minimum(vmn, sk), jnp.maximum(vmx, sk),
                        vneg + jnp.where(sk < 0, 1, 0).astype(jnp.int32))
            vmn, vmx, vneg = lax.fori_loop(
                0, nv, p0,
                (jnp.full((_L,), 2**31 - 1, jnp.int32),
                 jnp.full((_L,), -(2**31), jnp.int32),
                 jnp.zeros((_L,), jnp.int32)), unroll=4)
            mn, mx = vmn[0], vmx[0]
            for i in range(1, _L):
                mn = jnp.minimum(mn, vmn[i])
                mx = jnp.maximum(mx, vmx[i])
            cneg = hsum(vneg)

            x = mn ^ mx
            fx = x.astype(jnp.float32)
            e = (lax.bitcast_convert_type(fx, jnp.int32) >> 23) - 127
            mixed = x < 0
            h = jnp.where(mixed, jnp.int32(31), e)

            k0 = jnp.int32(_K)
            use_neg = mixed & (k0 <= cneg)
            s = jnp.clip(h + 1, 0, 31)
            mask_hi = jnp.where(h >= 31, jnp.int32(0), jnp.int32(-1) << s)
            prefix0 = jnp.where(
                mixed,
                jnp.where(use_neg, jnp.int32(-(2**31)), jnp.int32(0)),
                mn & mask_hi)
            krem0 = jnp.where(mixed & jnp.logical_not(use_neg),
                              k0 - cneg, k0)
            nbits = jnp.where(mixed, jnp.int32(31),
                              jnp.maximum(h + 1, 0))

            def vbody(i, carry):
                prefix, krem = carry
                b = nbits - 1 - i
                ps = prefix >> b
                cnt0 = count_pass(lambda v, j: (v >> b) == ps)
                take1 = krem > cnt0
                prefix = jnp.where(take1, prefix | (jnp.int32(1) << b),
                                   prefix)
                krem = jnp.where(take1, krem - cnt0, krem)
                return prefix, krem
            t, _unused = lax.fori_loop(0, nbits, vbody, (prefix0, krem0))

            def c2(j, carry):
                a_lt, a_eq = carry
                v = skey_v[pl.ds(j * _L, _L)]
                a_lt = a_lt + jnp.where(v < t, 1, 0).astype(jnp.int32)
                a_eq = a_eq + jnp.where(v == t, 1, 0).astype(jnp.int32)
                return a_lt, a_eq
            a_lt, a_eq = lax.fori_loop(
                0, nv, c2,
                (jnp.zeros((_L,), jnp.int32), jnp.zeros((_L,), jnp.int32)),
                unroll=4)
            cnt_lt = hsum(a_lt)
            cnt_eq = hsum(a_eq)
            need = k0 - cnt_lt

            def sel_idx(_op):
                def ib(i, carry):
                    ipref, nd = carry
                    b = jnp.int32(10) - i
                    ps = ipref >> b
                    cnt0 = count_pass(
                        lambda v, j: (v == t)
                        & (((iota + j * _L) >> b) == ps))
                    take1 = nd > cnt0
                    ipref = jnp.where(take1, ipref | (jnp.int32(1) << b),
                                      ipref)
                    nd = jnp.where(take1, nd - cnt0, nd)
                    return ipref, nd
                ipref, _nd = lax.fori_loop(0, 11, ib, (jnp.int32(0), need))
                return ipref
            ipref = lax.cond(cnt_eq == need,
                             lambda _op: jnp.int32(_N - 1), sel_idx, 0)

            def eb(j, _c):
                v = skey_v[pl.ds(j * _L, _L)]
                col = iota + j * _L
                sel = (v < t) | ((v == t) & (col <= ipref))
                out_v[pl.ds(ri * _N + j * _L, _L)] = jnp.where(
                    sel, jnp.float32(1.0), jnp.float32(0.0))
                return 0
            lax.fori_loop(0, nv, eb, 0, unroll=4)
            return 0

        def batch_body(bi, _):
            gbase = wbase + bi * batch
            pltpu.sync_copy(
                d2_hbm.at[pl.ds((row_base + gbase) * _N, batch * _N)], in_v)
            lax.fori_loop(0, batch, row_body, 0)
            pltpu.sync_copy(out_v, a_hbm.at[pl.ds(gbase * _N, batch * _N)])
            return 0
        lax.fori_loop(0, nb, batch_body, 0)

    return sc_kernel(d2_flat).reshape(s_rows, _N)


_S_SC = 512


def kernel(toLearn):
    flat = toLearn.reshape(toLearn.shape[0], -1)
    n = flat.shape[0]
    sq = jnp.sum(flat * flat, axis=1)
    d2 = sq[:, None] + sq[None, :] - 2.0 * (flat @ flat.T)
    d2 = d2 + jnp.eye(n, dtype=flat.dtype) * 1e12

    parts = []
    if _S_SC > 0:
        parts.append(_sc_knn_adjacency(d2.reshape(-1), _S_SC, 0))
    if _S_SC < n:
        parts.append(_knn_adjacency_tc(d2[_S_SC:]))
    A = parts[0] if len(parts) == 1 else jnp.concatenate(parts, axis=0)

    W = 0.5 * (A + A.T)
    deg = jnp.sum(W, axis=1)
    dd = jnp.sqrt(deg)
    L = jnp.eye(n, dtype=jnp.float32) - (W / dd[:, None]) / dd[None, :]
    evals, evecs = jnp.linalg.eigh(L)
    emb = evecs[:, 1:_NCOMP + 1] / dd[:, None]
    max_abs_row = jnp.argmax(jnp.abs(emb), axis=0)
    signs = jnp.sign(emb[max_abs_row, jnp.arange(emb.shape[1])])
    signs = jnp.where(signs == 0, 1.0, signs)
    emb = jax.lax.stop_gradient(emb * signs[None, :])
    return emb.reshape(n, 1, 28, 28).astype(jnp.float32)


# speedup vs baseline: 1.0170x; 1.0001x over previous
import functools

import jax
import jax.numpy as jnp
from jax import lax
from jax.experimental import pallas as pl
from jax.experimental.pallas import tpu as pltpu
from jax.experimental.pallas import tpu_sc as plsc

_N = 2048
_K = 150
_NCOMP = 784
_BLK = 256
_L = 16


def _knn_select_body(d2_ref, a_ref):
    nrow = d2_ref.shape[0]
    d2 = d2_ref[...]
    bits = jax.lax.bitcast_convert_type(d2, jnp.int32)
    skey = jnp.where(bits < 0, bits ^ jnp.int32(0x7FFFFFFF), bits)

    k0 = jnp.full((nrow, 1), _K, jnp.int32)
    cnt_neg = jnp.sum((skey < 0).astype(jnp.int32), axis=1, keepdims=True)
    use_neg = k0 <= cnt_neg
    prefix = jnp.where(use_neg, jnp.int32(-(2**31)), jnp.int32(0))
    k_rem = jnp.where(use_neg, k0, k0 - cnt_neg)

    for b in range(30, -1, -1):
        match0 = (skey >> b) == (prefix >> b)
        cnt0 = jnp.sum(match0.astype(jnp.int32), axis=1, keepdims=True)
        take1 = k_rem > cnt0
        prefix = jnp.where(take1, prefix | jnp.int32(1 << b), prefix)
        k_rem = jnp.where(take1, k_rem - cnt0, k_rem)

    t = prefix
    lt = skey < t
    tie = skey == t
    need = k0 - jnp.sum(lt.astype(jnp.int32), axis=1, keepdims=True)
    col = jax.lax.broadcasted_iota(jnp.int32, (nrow, _N), 1)
    ipref = jnp.zeros((nrow, 1), jnp.int32)
    for b in range(10, -1, -1):
        m0 = tie & ((col >> b) == (ipref >> b))
        cnt0 = jnp.sum(m0.astype(jnp.int32), axis=1, keepdims=True)
        take1 = need > cnt0
        ipref = jnp.where(take1, ipref | jnp.int32(1 << b), ipref)
        need = jnp.where(take1, need - cnt0, need)

    sel = lt | (tie & (col <= ipref))
    a_ref[...] = sel.astype(jnp.float32)


def _knn_adjacency_tc(d2_rows, interpret=False):
    nrows = d2_rows.shape[0]
    blk = min(_BLK, nrows)
    return pl.pallas_call(
        _knn_select_body,
        grid=(nrows // blk,),
        in_specs=[pl.BlockSpec((blk, _N), lambda i: (i, 0))],
        out_specs=pl.BlockSpec((blk, _N), lambda i: (i, 0)),
        out_shape=jax.ShapeDtypeStruct((nrows, _N), jnp.float32),
        interpret=interpret,
    )(d2_rows)


def _sc_knn_adjacency(d2_flat, s_rows, row_base):
    info = plsc.get_sparse_core_info()
    nw = info.num_cores * info.num_subcores
    rp = s_rows // nw
    batch = min(16, rp)
    nb = rp // batch
    nv = _N // _L

    mesh = plsc.VectorSubcoreMesh(core_axis_name="c", subcore_axis_name="s")

    @functools.partial(
        pl.kernel, mesh=mesh,
        out_type=jax.ShapeDtypeStruct((s_rows * _N,), jnp.float32),
        scratch_types=[
            pltpu.VMEM((batch * _N,), jnp.float32),
            pltpu.VMEM((batch * _N,), jnp.float32),
            pltpu.VMEM((_N,), jnp.int32),
        ],
    )
    def sc_kernel(d2_hbm, a_hbm, in_v, out_v, skey_v):
        wid = lax.axis_index("s") * info.num_cores + lax.axis_index("c")
        wbase = wid * rp
        iota = lax.broadcasted_iota(jnp.int32, (_L,), 0)

        def hsum(v):
            parts = [v[i] for i in range(_L)]
            while len(parts) > 1:
                parts = [parts[i] + parts[i + 1]
                         for i in range(0, len(parts), 2)]
            return parts[0]

        def count_pass(pred):
            def cb(j, acc):
                v = skey_v[pl.ds(j * _L, _L)]
                return acc + jnp.where(pred(v, j), 1, 0).astype(jnp.int32)
            acc = lax.fori_loop(0, nv, cb, jnp.zeros((_L,), jnp.int32),
                                unroll=4)
            return hsum(acc)

        def make_row_body(gbase):
          def row_body(ri, _):
            dcol = row_base + gbase + ri

            def p0(j, carry):
                vmn, vmx, vneg = carry
                v = in_v[pl.ds(ri * _N + j * _L, _L)]
                b = lax.bitcast_convert_type(v, jnp.int32)
                sk = jnp.where(b < 0, b ^ jnp.int32(0x7FFFFFFF), b)
                skey_v[pl.ds(j * _L, _L)] = sk
                sk_nd = jnp.where((iota + j * _L) == dcol,
                                  jnp.int32(-(2**31)), sk)
                return (jnp.minimum(vmn, sk), jnp.maximum(vmx, sk_nd),
                        vneg + jnp.where(sk < 0, 1, 0).astype(jnp.int32))
            vmn, vmx, vneg = lax.fori_loop(
                0, nv, p0,
                (jnp.full((_L,), 2**31 - 1, jnp.int32),
                 jnp.full((_L,), -(2**31), jnp.int32),
                 jnp.zeros((_L,), jnp.int32)), unroll=4)
            mn, mx = vmn[0], vmx[0]
            for i in range(1, _L):
                mn = jnp.minimum(mn, vmn[i])
                mx = jnp.maximum(mx, vmx[i])
            cneg = hsum(vneg)

            x = mn ^ mx
            fx = x.astype(jnp.float32)
            e = (lax.bitcast_convert_type(fx, jnp.int32) >> 23) - 127
            mixed = x < 0
            h = jnp.where(mixed, jnp.int32(31), e)

            k0 = jnp.int32(_K)
            use_neg = mixed & (k0 <= cneg)
            s = jnp.clip(h + 1, 0, 31)
            mask_hi = jnp.where(h >= 31, jnp.int32(0), jnp.int32(-1) << s)
            prefix0 = jnp.where(
                mixed,
                jnp.where(use_neg, jnp.int32(-(2**31)), jnp.int32(0)),
                mn & mask_hi)
            krem0 = jnp.where(mixed & jnp.logical_not(use_neg),
                              k0 - cneg, k0)
            nbits = jnp.where(mixed, jnp.int32(31),
                              jnp.maximum(h + 1, 0))

            def vbody(i, carry):
                prefix, krem = carry
                b = nbits - 1 - i
                ps = prefix >> b
                cnt0 = count_pass(lambda v, j: (v >> b) == ps)
                take1 = krem > cnt0
                prefix = jnp.where(take1, prefix | (jnp.int32(1) << b),
                                   prefix)
                krem = jnp.where(take1, krem - cnt0, krem)
                return prefix, krem
            t, _unused = lax.fori_loop(0, nbits, vbody, (prefix0, krem0))

            def c2(j, carry):
                a_lt, a_eq = carry
                v = skey_v[pl.ds(j * _L, _L)]
                a_lt = a_lt + jnp.where(v < t, 1, 0).astype(jnp.int32)
                a_eq = a_eq + jnp.where(v == t, 1, 0).astype(jnp.int32)
                return a_lt, a_eq
            a_lt, a_eq = lax.fori_loop(
                0, nv, c2,
                (jnp.zeros((_L,), jnp.int32), jnp.zeros((_L,), jnp.int32)),
                unroll=4)
            cnt_lt = hsum(a_lt)
            cnt_eq = hsum(a_eq)
            need = k0 - cnt_lt

            def sel_idx(_op):
                def ib(i, carry):
                    ipref, nd = carry
                    b = jnp.int32(10) - i
                    ps = ipref >> b
                    cnt0 = count_pass(
                        lambda v, j: (v == t)
                        & (((iota + j * _L) >> b) == ps))
                    take1 = nd > cnt0
                    ipref = jnp.where(take1, ipref | (jnp.int32(1) << b),
                                      ipref)
                    nd = jnp.where(take1, nd - cnt0, nd)
                    return ipref, nd
                ipref, _nd = lax.fori_loop(0, 11, ib, (jnp.int32(0), need))
                return ipref
            ipref = lax.cond(cnt_eq == need,
                             lambda _op: jnp.int32(_N - 1), sel_idx, 0)

            def eb(j, _c):
                v = skey_v[pl.ds(j * _L, _L)]
                col = iota + j * _L
                sel = (v < t) | ((v == t) & (col <= ipref))
                out_v[pl.ds(ri * _N + j * _L, _L)] = jnp.where(
                    sel, jnp.float32(1.0), jnp.float32(0.0))
                return 0
            lax.fori_loop(0, nv, eb, 0, unroll=4)
            return 0
          return row_body

        def batch_body(bi, _):
            gbase = wbase + bi * batch
            pltpu.sync_copy(
                d2_hbm.at[pl.ds((row_base + gbase) * _N, batch * _N)], in_v)
            lax.fori_loop(0, batch, make_row_body(gbase), 0)
            pltpu.sync_copy(out_v, a_hbm.at[pl.ds(gbase * _N, batch * _N)])
            return 0
        lax.fori_loop(0, nb, batch_body, 0)

    return sc_kernel(d2_flat).reshape(s_rows, _N)


_S_SC = 512


def kernel(toLearn):
    flat = toLearn.reshape(toLearn.shape[0], -1)
    n = flat.shape[0]
    sq = jnp.sum(flat * flat, axis=1)
    d2 = sq[:, None] + sq[None, :] - 2.0 * (flat @ flat.T)
    d2 = d2 + jnp.eye(n, dtype=flat.dtype) * 1e12

    parts = []
    if _S_SC > 0:
        parts.append(_sc_knn_adjacency(d2.reshape(-1), _S_SC, 0))
    if _S_SC < n:
        parts.append(_knn_adjacency_tc(d2[_S_SC:]))
    A = parts[0] if len(parts) == 1 else jnp.concatenate(parts, axis=0)

    W = 0.5 * (A + A.T)
    deg = jnp.sum(W, axis=1)
    dd = jnp.sqrt(deg)
    L = jnp.eye(n, dtype=jnp.float32) - (W / dd[:, None]) / dd[None, :]
    evals, evecs = jnp.linalg.eigh(L)
    emb = evecs[:, 1:_NCOMP + 1] / dd[:, None]
    max_abs_row = jnp.argmax(jnp.abs(emb), axis=0)
    signs = jnp.sign(emb[max_abs_row, jnp.arange(emb.shape[1])])
    signs = jnp.where(signs == 0, 1.0, signs)
    emb = jax.lax.stop_gradient(emb * signs[None, :])
    return emb.reshape(n, 1, 28, 28).astype(jnp.float32)
